# baseline (speedup 1.0000x reference)
import functools

import jax
import jax.numpy as jnp
from jax.experimental import pallas as pl
from jax.experimental.pallas import tpu as pltpu


def _cell_body(x_ref, nh_ref, nc_ref, wiou_ref, biou_ref, wf_ref, bf_ref,
               uf_ref, buf_ref, uiou_ref, buiou_ref, h_ref, c_ref,
               *, n_ch, h_size):
    x = x_ref[...]
    b = x.shape[0]

    f_x = (jnp.dot(x, wf_ref[...], preferred_element_type=jnp.float32)
           + bf_ref[...] + buf_ref[...])

    nh = nh_ref[...]
    nc = nc_ref[...]

    nh2 = nh.reshape(b * n_ch, h_size)
    f_h = jnp.dot(nh2, uf_ref[...],
                  preferred_element_type=jnp.float32).reshape(b, n_ch, h_size)
    f = jax.nn.sigmoid(f_h + f_x[:, None, :])
    c_aggr = jnp.sum(f * nc, axis=1)

    hsum = jnp.sum(nh, axis=1)
    iou = (jnp.dot(x, wiou_ref[...], preferred_element_type=jnp.float32)
           + biou_ref[...]
           + jnp.dot(hsum, uiou_ref[...], preferred_element_type=jnp.float32)
           + n_ch * buiou_ref[...])

    i = jax.nn.sigmoid(iou[:, :h_size])
    o = jax.nn.sigmoid(iou[:, h_size:2 * h_size])
    u = jnp.tanh(iou[:, 2 * h_size:])
    c = i * u + c_aggr
    c_ref[...] = c
    h_ref[...] = o * jnp.tanh(c)


def kernel(x, n_h, n_c, W_iou_x, b_iou_x, W_f_x, b_f_x, U_f, b_U_f, U_iou,
           b_U_iou):
    n, n_ch, h_size = n_h.shape
    x_size = x.shape[1]
    block = 200
    assert n % block == 0
    grid = (n // block,)

    biou2 = b_iou_x.reshape(1, -1)
    bf2 = b_f_x.reshape(1, -1)
    buf2 = b_U_f.reshape(1, -1)
    buiou2 = b_U_iou.reshape(1, -1)

    fixed = lambda *shape: pl.BlockSpec(shape, lambda i: (0,) * len(shape))

    h_out, c_out = pl.pallas_call(
        functools.partial(_cell_body, n_ch=n_ch, h_size=h_size),
        grid=grid,
        in_specs=[
            pl.BlockSpec((block, x_size), lambda i: (i, 0)),
            pl.BlockSpec((block, n_ch, h_size), lambda i: (i, 0, 0)),
            pl.BlockSpec((block, n_ch, h_size), lambda i: (i, 0, 0)),
            fixed(x_size, 3 * h_size),
            fixed(1, 3 * h_size),
            fixed(x_size, h_size),
            fixed(1, h_size),
            fixed(h_size, h_size),
            fixed(1, h_size),
            fixed(h_size, 3 * h_size),
            fixed(1, 3 * h_size),
        ],
        out_specs=[
            pl.BlockSpec((block, h_size), lambda i: (i, 0)),
            pl.BlockSpec((block, h_size), lambda i: (i, 0)),
        ],
        out_shape=[
            jax.ShapeDtypeStruct((n, h_size), jnp.float32),
            jax.ShapeDtypeStruct((n, h_size), jnp.float32),
        ],
        compiler_params=pltpu.CompilerParams(
            dimension_semantics=("parallel",),
            vmem_limit_bytes=100 * 1024 * 1024),
    )(x, n_h, n_c, W_iou_x, biou2, W_f_x, bf2, U_f, buf2, U_iou, buiou2)
    return h_out, c_out


# speedup vs baseline: 1.1172x; 1.1172x over previous
import functools

import jax
import jax.numpy as jnp
from jax.experimental import pallas as pl
from jax.experimental.pallas import tpu as pltpu


def _cell_body(x_ref, nh_ref, nc_ref, wiou_ref, biou_ref, wf_ref, bf_ref,
               uf_ref, buf_ref, uiou_ref, buiou_ref, h_ref, c_ref,
               *, n_ch, h_size):
    x = x_ref[:, 0, :]
    b = x.shape[0]

    f_x = (jnp.dot(x, wf_ref[...], preferred_element_type=jnp.float32)
           + bf_ref[...] + buf_ref[...])

    nh = nh_ref[...]
    nc = nc_ref[...]

    nh2 = nh.reshape(b * n_ch, h_size)
    f_h = jnp.dot(nh2, uf_ref[...],
                  preferred_element_type=jnp.float32).reshape(b, n_ch, h_size)
    f = jax.nn.sigmoid(f_h + f_x[:, None, :])
    c_aggr = jnp.sum(f * nc, axis=1)

    hsum = jnp.sum(nh, axis=1)
    iou = (jnp.dot(x, wiou_ref[...], preferred_element_type=jnp.float32)
           + biou_ref[...]
           + jnp.dot(hsum, uiou_ref[...], preferred_element_type=jnp.float32)
           + n_ch * buiou_ref[...])

    i = jax.nn.sigmoid(iou[:, :h_size])
    o = jax.nn.sigmoid(iou[:, h_size:2 * h_size])
    u = jnp.tanh(iou[:, 2 * h_size:])
    c = i * u + c_aggr
    c_ref[:, 0, :] = c
    h_ref[:, 0, :] = o * jnp.tanh(c)


def kernel(x, n_h, n_c, W_iou_x, b_iou_x, W_f_x, b_f_x, U_f, b_U_f, U_iou,
           b_U_iou):
    n, n_ch, h_size = n_h.shape
    x_size = x.shape[1]
    block = 500
    assert n % block == 0
    grid = (n // block,)

    biou2 = b_iou_x.reshape(1, -1)
    bf2 = b_f_x.reshape(1, -1)
    buf2 = b_U_f.reshape(1, -1)
    buiou2 = b_U_iou.reshape(1, -1)

    fixed = lambda *shape: pl.BlockSpec(shape, lambda i: (0,) * len(shape))

    h_out, c_out = pl.pallas_call(
        functools.partial(_cell_body, n_ch=n_ch, h_size=h_size),
        grid=grid,
        in_specs=[
            pl.BlockSpec((block, 1, x_size), lambda i: (i, 0, 0)),
            pl.BlockSpec((block, n_ch, h_size), lambda i: (i, 0, 0)),
            pl.BlockSpec((block, n_ch, h_size), lambda i: (i, 0, 0)),
            fixed(x_size, 3 * h_size),
            fixed(1, 3 * h_size),
            fixed(x_size, h_size),
            fixed(1, h_size),
            fixed(h_size, h_size),
            fixed(1, h_size),
            fixed(h_size, 3 * h_size),
            fixed(1, 3 * h_size),
        ],
        out_specs=[
            pl.BlockSpec((block, 1, h_size), lambda i: (i, 0, 0)),
            pl.BlockSpec((block, 1, h_size), lambda i: (i, 0, 0)),
        ],
        out_shape=[
            jax.ShapeDtypeStruct((n, 1, h_size), jnp.float32),
            jax.ShapeDtypeStruct((n, 1, h_size), jnp.float32),
        ],
        compiler_params=pltpu.CompilerParams(
            dimension_semantics=("parallel",),
            vmem_limit_bytes=100 * 1024 * 1024),
    )(x.reshape(n, 1, x_size), n_h, n_c, W_iou_x, biou2, W_f_x, bf2, U_f,
      buf2, U_iou, buiou2)
    return h_out.reshape(n, h_size), c_out.reshape(n, h_size)


# speedup vs baseline: 1.1286x; 1.0102x over previous
import functools

import jax
import jax.numpy as jnp
from jax.experimental import pallas as pl
from jax.experimental.pallas import tpu as pltpu


def _sigmoid(z):
    return 0.5 * jnp.tanh(0.5 * z) + 0.5


def _cell_body(x_ref, nh_ref, nc_ref, wiou_ref, biou_ref, wf_ref, bf_ref,
               uf_ref, buf_ref, uiou_ref, buiou_ref, h_ref, c_ref,
               *, n_ch, h_size):
    x = x_ref[...]
    b = x.shape[0]

    f_x = (jnp.dot(x, wf_ref[...], preferred_element_type=jnp.float32)
           + bf_ref[...] + buf_ref[...])

    nh = nh_ref[...]
    nc = nc_ref[...]

    nh2 = nh.reshape(b * n_ch, h_size)
    f_h = jnp.dot(nh2, uf_ref[...],
                  preferred_element_type=jnp.float32).reshape(b, n_ch, h_size)
    f = _sigmoid(f_h + f_x[:, None, :])
    c_aggr = jnp.sum(f * nc, axis=1)

    hsum = jnp.sum(nh, axis=1)
    iou = (jnp.dot(x, wiou_ref[...], preferred_element_type=jnp.float32)
           + biou_ref[...]
           + jnp.dot(hsum, uiou_ref[...], preferred_element_type=jnp.float32)
           + n_ch * buiou_ref[...])

    i = _sigmoid(iou[:, :h_size])
    o = _sigmoid(iou[:, h_size:2 * h_size])
    u = jnp.tanh(iou[:, 2 * h_size:])
    c = i * u + c_aggr
    c_ref[...] = c
    h_ref[...] = o * jnp.tanh(c)


def kernel(x, n_h, n_c, W_iou_x, b_iou_x, W_f_x, b_f_x, U_f, b_U_f, U_iou,
           b_U_iou):
    n, n_ch, h_size = n_h.shape
    x_size = x.shape[1]
    block = 400
    assert n % block == 0
    grid = (n // block,)

    biou2 = b_iou_x.reshape(1, -1)
    bf2 = b_f_x.reshape(1, -1)
    buf2 = b_U_f.reshape(1, -1)
    buiou2 = b_U_iou.reshape(1, -1)

    fixed = lambda *shape: pl.BlockSpec(shape, lambda i: (0,) * len(shape))

    h_out, c_out = pl.pallas_call(
        functools.partial(_cell_body, n_ch=n_ch, h_size=h_size),
        grid=grid,
        in_specs=[
            pl.BlockSpec((block, x_size), lambda i: (i, 0)),
            pl.BlockSpec((block, n_ch, h_size), lambda i: (i, 0, 0)),
            pl.BlockSpec((block, n_ch, h_size), lambda i: (i, 0, 0)),
            fixed(x_size, 3 * h_size),
            fixed(1, 3 * h_size),
            fixed(x_size, h_size),
            fixed(1, h_size),
            fixed(h_size, h_size),
            fixed(1, h_size),
            fixed(h_size, 3 * h_size),
            fixed(1, 3 * h_size),
        ],
        out_specs=[
            pl.BlockSpec((block, h_size), lambda i: (i, 0)),
            pl.BlockSpec((block, h_size), lambda i: (i, 0)),
        ],
        out_shape=[
            jax.ShapeDtypeStruct((n, h_size), jnp.float32),
            jax.ShapeDtypeStruct((n, h_size), jnp.float32),
        ],
        compiler_params=pltpu.CompilerParams(
            dimension_semantics=("parallel",)),
    )(x, n_h, n_c, W_iou_x, biou2, W_f_x, bf2, U_f, buf2, U_iou, buiou2)
    return h_out, c_out
